# baseline (speedup 1.0000x reference)
import functools

import numpy as _np
import jax
import jax.numpy as jnp
from jax import lax
from jax.experimental import pallas as pl
from jax.experimental.pallas import tpu as pltpu
from jax.experimental.pallas import tpu_sc as plsc

NUM_CORES = 2
NUM_SUBCORES = 16
LANES = 16
NW = NUM_CORES * NUM_SUBCORES

N_POS = 1024 * 16 * 16
D = 32
ROWS = 300
PER_TEC = N_POS // NW
CHUNK = 1024
N_CHUNKS = PER_TEC // CHUNK
GROUPS = CHUNK // LANES

_mesh = plsc.VectorSubcoreMesh(
    core_axis_name="c", subcore_axis_name="s",
    num_cores=NUM_CORES, num_subcores=NUM_SUBCORES,
)


@functools.partial(
    pl.kernel,
    out_type=jax.ShapeDtypeStruct((N_POS * D,), jnp.float32),
    mesh=_mesh,
    scratch_types=[
        pltpu.VMEM((ROWS * D,), jnp.float32),
        pltpu.VMEM((LANES * LANES,), jnp.int32),
        pltpu.VMEM((CHUNK,), jnp.int32),
        pltpu.VMEM((CHUNK,), jnp.int32),
        pltpu.VMEM((CHUNK * D,), jnp.float32),
        pltpu.VMEM((CHUNK * D,), jnp.float32),
        pltpu.SemaphoreType.DMA,
        pltpu.SemaphoreType.DMA,
        pltpu.SemaphoreType.DMA,
        pltpu.SemaphoreType.DMA,
    ],
    compiler_params=pltpu.CompilerParams(needs_layout_passes=False),
)
def _sc_encode(x_hbm, table_hbm, diag_hbm, out_hbm, table_v, diag_v,
               x_v0, x_v1, out_v0, out_v1, sx0, sx1, so0, so1):
    wid = lax.axis_index("s") * NUM_CORES + lax.axis_index("c")
    pltpu.sync_copy(table_hbm, table_v)
    pltpu.sync_copy(diag_hbm, diag_v)
    lane = lax.iota(jnp.int32, LANES)
    x_bufs = (x_v0, x_v1)
    out_bufs = (out_v0, out_v1)
    x_sems = (sx0, sx1)
    out_sems = (so0, so1)

    def start_x(chunk, b):
        base = wid * PER_TEC + chunk * CHUNK
        pltpu.async_copy(x_hbm.at[pl.ds(base, CHUNK)], x_bufs[b], x_sems[b])

    def wait_x(b):
        pltpu.make_async_copy(
            x_hbm.at[pl.ds(0, CHUNK)], x_bufs[b], x_sems[b]).wait()

    def start_out(chunk, b):
        base = wid * PER_TEC + chunk * CHUNK
        pltpu.async_copy(
            out_bufs[b], out_hbm.at[pl.ds(base * D, CHUNK * D)], out_sems[b])

    def wait_out(b):
        pltpu.make_async_copy(
            out_bufs[b], out_hbm.at[pl.ds(0, CHUNK * D)], out_sems[b]).wait()

    def compute(b):
        x_v, out_v = x_bufs[b], out_bufs[b]

        @plsc.parallel_loop(0, GROUPS, unroll=4)
        def group_body(g):
            goff = g * LANES
            xp = x_v[pl.ds(goff, LANES)]
            i2 = xp >> 20
            r = xp - (i2 << 20)
            i1 = r >> 10
            i0 = r - (i1 << 10)
            b0 = i0 * D
            b1 = i1 * D
            b2 = i2 * D
            ob = (goff + lane) * D
            for k in range(LANES):
                dva = diag_v[pl.ds(k * LANES, LANES)]
                dvb = dva + LANES
                sa = (plsc.load_gather(table_v, [b0 + dva])
                      + plsc.load_gather(table_v, [b1 + dva])
                      + plsc.load_gather(table_v, [b2 + dva]))
                plsc.store_scatter(out_v, [ob + dva], sa)
                sb = (plsc.load_gather(table_v, [b0 + dvb])
                      + plsc.load_gather(table_v, [b1 + dvb])
                      + plsc.load_gather(table_v, [b2 + dvb]))
                plsc.store_scatter(out_v, [ob + dvb], sb)

    start_x(0, 0)
    start_x(1, 1)

    def pair_body(p, carry):
        c0 = p * 2

        def half(b):
            wait_x(b)

            @pl.when(p > 0)
            def _():
                wait_out(b)

            compute(b)
            start_out(c0 + b, b)

            @pl.when(p < (N_CHUNKS // 2) - 1)
            def _():
                start_x(c0 + 2 + b, b)

        half(0)
        half(1)
        return carry

    lax.fori_loop(0, N_CHUNKS // 2, pair_body, 0)
    wait_out(0)
    wait_out(1)


_DIAG = _np.asarray(
    [(l + k) % LANES for k in range(LANES) for l in range(LANES)],
    dtype=_np.int32)


_PACK_W = _np.asarray([1, 1 << 10, 1 << 20], dtype=_np.int32)
_PACK_C = _np.int32((100 << 10) + (200 << 20))


def kernel(x, table):
    xp = (x.reshape(-1, 3) * _PACK_W).sum(-1) + _PACK_C
    return _sc_encode(xp, table.reshape(-1), jnp.asarray(_DIAG))


# speedup vs baseline: 1.2323x; 1.2323x over previous
import functools

import numpy as _np
import jax
import jax.numpy as jnp
from jax import lax
from jax.experimental import pallas as pl
from jax.experimental.pallas import tpu as pltpu
from jax.experimental.pallas import tpu_sc as plsc

NUM_CORES = 2
NUM_SUBCORES = 16
LANES = 16
NW = NUM_CORES * NUM_SUBCORES

N_POS = 1024 * 16 * 16
D = 32
ROWS = 300
PER_TEC = N_POS // NW
CHUNK = 1024
N_CHUNKS = PER_TEC // CHUNK
GROUPS = CHUNK // LANES

_mesh = plsc.VectorSubcoreMesh(
    core_axis_name="c", subcore_axis_name="s",
    num_cores=NUM_CORES, num_subcores=NUM_SUBCORES,
)


@functools.partial(
    pl.kernel,
    out_type=jax.ShapeDtypeStruct((N_POS * D,), jnp.float32),
    mesh=_mesh,
    scratch_types=[
        pltpu.VMEM((ROWS * D,), jnp.float32),
        pltpu.VMEM((LANES * LANES,), jnp.int32),
        pltpu.VMEM((CHUNK,), jnp.int32),
        pltpu.VMEM((CHUNK,), jnp.int32),
        pltpu.VMEM((CHUNK * D,), jnp.float32),
        pltpu.VMEM((CHUNK * D,), jnp.float32),
        pltpu.SemaphoreType.DMA,
        pltpu.SemaphoreType.DMA,
        pltpu.SemaphoreType.DMA,
        pltpu.SemaphoreType.DMA,
    ],
    compiler_params=pltpu.CompilerParams(needs_layout_passes=False),
)
def _sc_encode(x_hbm, table_hbm, diag_hbm, out_hbm, table_v, diag_v,
               x_v0, x_v1, out_v0, out_v1, sx0, sx1, so0, so1):
    wid = lax.axis_index("s") * NUM_CORES + lax.axis_index("c")
    pltpu.sync_copy(table_hbm, table_v)
    pltpu.sync_copy(diag_hbm, diag_v)
    lane = lax.iota(jnp.int32, LANES)
    x_bufs = (x_v0, x_v1)
    out_bufs = (out_v0, out_v1)
    x_sems = (sx0, sx1)
    out_sems = (so0, so1)

    def start_x(chunk, b):
        base = wid * PER_TEC + chunk * CHUNK
        pltpu.async_copy(x_hbm.at[pl.ds(base, CHUNK)], x_bufs[b], x_sems[b])

    def wait_x(b):
        pltpu.make_async_copy(
            x_hbm.at[pl.ds(0, CHUNK)], x_bufs[b], x_sems[b]).wait()

    def start_out(chunk, b):
        base = wid * PER_TEC + chunk * CHUNK
        pltpu.async_copy(
            out_bufs[b], out_hbm.at[pl.ds(base * D, CHUNK * D)], out_sems[b])

    def wait_out(b):
        pltpu.make_async_copy(
            out_bufs[b], out_hbm.at[pl.ds(0, CHUNK * D)], out_sems[b]).wait()

    def compute(b):
        x_v, out_v = x_bufs[b], out_bufs[b]

        @plsc.parallel_loop(0, GROUPS, unroll=2)
        def group_body(g):
            goff = g * LANES
            xp = x_v[pl.ds(goff, LANES)]
            i2 = xp >> 20
            r = xp - (i2 << 20)
            i1 = r >> 10
            i0 = r - (i1 << 10)
            b0 = i0 * D
            b1 = i1 * D
            b2 = i2 * D
            ob = (goff + lane) * D
            for k in range(LANES):
                dva = diag_v[pl.ds(k * LANES, LANES)]
                dvb = dva + LANES
                sa = (plsc.load_gather(table_v, [b0 + dva])
                      + plsc.load_gather(table_v, [b1 + dva])
                      + plsc.load_gather(table_v, [b2 + dva]))
                plsc.store_scatter(out_v, [ob + dva], sa)
                sb = (plsc.load_gather(table_v, [b0 + dvb])
                      + plsc.load_gather(table_v, [b1 + dvb])
                      + plsc.load_gather(table_v, [b2 + dvb]))
                plsc.store_scatter(out_v, [ob + dvb], sb)

    start_x(0, 0)
    start_x(1, 1)

    def pair_body(p, carry):
        c0 = p * 2

        def half(b):
            wait_x(b)

            @pl.when(p > 0)
            def _():
                wait_out(b)

            compute(b)
            start_out(c0 + b, b)

            @pl.when(p < (N_CHUNKS // 2) - 1)
            def _():
                start_x(c0 + 2 + b, b)

        half(0)
        half(1)
        return carry

    lax.fori_loop(0, N_CHUNKS // 2, pair_body, 0)
    wait_out(0)
    wait_out(1)


_DIAG = _np.asarray(
    [(l + k) % LANES for k in range(LANES) for l in range(LANES)],
    dtype=_np.int32)


_PACK_W = _np.asarray([1, 1 << 10, 1 << 20], dtype=_np.int32)
_PACK_C = _np.int32((100 << 10) + (200 << 20))


def kernel(x, table):
    xp = (x.reshape(-1, 3) * _PACK_W).sum(-1) + _PACK_C
    return _sc_encode(xp, table.reshape(-1), jnp.asarray(_DIAG))
